# baseline (speedup 1.0000x reference)
import jax
import jax.numpy as jnp
from jax import lax
from jax.experimental import pallas as pl
from jax.experimental.pallas import tpu as pltpu
from jax.experimental.pallas import tpu_sc as plsc

SEQ = 200
DIM = 64
ROWS = SEQ * DIM
NC = 2
NS = 16
NW = NC * NS
LAN = 16
W = 128
CSUB = 128
CRL = 4


def _sc_body(ex_hbm, cat_hbm, pos_hbm, out_hbm, ex_v, cat_v, out_v, pos_v, sem_a, sem_b):
    nsub = ex_hbm.shape[0]
    sub_per_row = nsub // ROWS
    chunks = nsub // CSUB
    cpw = chunks // NW
    wid = lax.axis_index("s") * NC + lax.axis_index("c")

    def chunk_body(i, _):
        c = wid * cpw + i
        s0 = c * CSUB
        src_ex = ex_hbm.at[pl.ds(s0, CSUB)]
        src_cat = cat_hbm.at[pl.ds(s0, CSUB)]
        cp_ex = pltpu.async_copy(src_ex, ex_v, sem_a)
        cp_cat = pltpu.async_copy(src_cat, cat_v, sem_b)
        pltpu.sync_copy(pos_hbm.at[pl.ds(c * CRL * LAN, CRL * LAN)], pos_v)
        cp_ex.wait()
        cp_cat.wait()
        for r in range(CRL):
            splat = pos_v[pl.ds(r * LAN, LAN)]

            def sub_body(q, _):
                row = r * sub_per_row + q

                def lane_body(j, _):
                    sl = pl.ds(j * LAN, LAN)
                    out_v[row, sl] = ex_v[row, sl] + cat_v[row, sl] + splat
                    return _

                lax.fori_loop(0, W // LAN, lane_body, 0)
                return _

            lax.fori_loop(0, sub_per_row, sub_body, 0)
        pltpu.sync_copy(out_v, out_hbm.at[pl.ds(s0, CSUB)])
        return _

    lax.fori_loop(0, cpw, chunk_body, 0)


def kernel(exercises, categories, position_embed):
    B = exercises.shape[0]
    nsub = ROWS * B // W
    ex2 = jnp.transpose(exercises, (1, 2, 0)).reshape(nsub, W)
    cat2 = jnp.transpose(categories, (1, 2, 0)).reshape(nsub, W)
    pos_rep = jnp.broadcast_to(
        position_embed.reshape(ROWS)[:, None], (ROWS, LAN)
    ).reshape(ROWS * LAN)
    mesh = plsc.VectorSubcoreMesh(core_axis_name="c", subcore_axis_name="s")
    sc_add = pl.kernel(
        _sc_body,
        out_type=jax.ShapeDtypeStruct((nsub, W), jnp.float32),
        mesh=mesh,
        scratch_types=[
            pltpu.VMEM((CSUB, W), jnp.float32),
            pltpu.VMEM((CSUB, W), jnp.float32),
            pltpu.VMEM((CSUB, W), jnp.float32),
            pltpu.VMEM((CRL * LAN,), jnp.float32),
            pltpu.SemaphoreType.DMA,
            pltpu.SemaphoreType.DMA,
        ],
    )
    out2 = sc_add(ex2, cat2, pos_rep)
    return jnp.transpose(out2.reshape(SEQ, DIM, B), (2, 0, 1))


# speedup vs baseline: 5.6299x; 5.6299x over previous
import jax
import jax.numpy as jnp
from jax.experimental import pallas as pl
from jax.experimental.pallas import tpu as pltpu

SEQ = 200
DIM = 64
BS = 8
BL = 2048


def _add_kernel(ex_ref, cat_ref, pos_ref, out_ref):
    out_ref[:] = ex_ref[:] + cat_ref[:] + pos_ref[:][:, :, None]


def kernel(exercises, categories, position_embed):
    B = exercises.shape[0]
    ex_t = jnp.transpose(exercises, (1, 2, 0))
    cat_t = jnp.transpose(categories, (1, 2, 0))
    out_t = pl.pallas_call(
        _add_kernel,
        grid=(SEQ // BS, B // BL),
        in_specs=[
            pl.BlockSpec((BS, DIM, BL), lambda i, j: (i, 0, j)),
            pl.BlockSpec((BS, DIM, BL), lambda i, j: (i, 0, j)),
            pl.BlockSpec((BS, DIM), lambda i, j: (i, 0)),
        ],
        out_specs=pl.BlockSpec((BS, DIM, BL), lambda i, j: (i, 0, j)),
        out_shape=jax.ShapeDtypeStruct((SEQ, DIM, B), jnp.float32),
        compiler_params=pltpu.CompilerParams(
            dimension_semantics=("arbitrary", "arbitrary"),
        ),
    )(ex_t, cat_t, position_embed)
    return jnp.transpose(out_t, (2, 0, 1))
